# baseline (speedup 1.0000x reference)
import functools

import jax
import jax.numpy as jnp
from jax import lax
from jax.experimental import pallas as pl
from jax.experimental.layout import Format, Layout
from jax.experimental.pallas import tpu as pltpu
from jax.experimental.pallas import tpu_sc as plsc

EMBED_DIM = 64
SCALE = float(EMBED_DIM) ** 0.5

NUM_CORES = 2
NUM_SUBCORES = 16
NW = NUM_CORES * NUM_SUBCORES

HALF = 100
HALFP = 104
CHUNK = 2 * HALFP
NBUF = 2


def _body(rows_per_w, ids_hbm, table_hbm, out_hbm, idx_v, bufs_g, bufs_s,
          sems_g, sems_s):
  wid = lax.axis_index("s") * NUM_CORES + lax.axis_index("c")
  nchunk = rows_per_w
  nouter = nchunk // NBUF

  pltpu.sync_copy(ids_hbm.at[wid], idx_v)

  def gathers(c, b):
    pltpu.async_copy(table_hbm.at[idx_v.at[c, pl.ds(0, HALFP)]],
                     bufs_g[b].at[pl.ds(0, HALFP)], sems_g[b])
    pltpu.async_copy(table_hbm.at[idx_v.at[c, pl.ds(HALFP, HALFP)]],
                     bufs_g[b].at[pl.ds(HALFP, HALFP)], sems_g[b])

  def gathers_wait(c, b):
    pltpu.make_async_copy(table_hbm.at[idx_v.at[c, pl.ds(0, HALFP)]],
                          bufs_g[b].at[pl.ds(0, HALFP)], sems_g[b]).wait()
    pltpu.make_async_copy(table_hbm.at[idx_v.at[c, pl.ds(HALFP, HALFP)]],
                          bufs_g[b].at[pl.ds(HALFP, HALFP)], sems_g[b]).wait()

  def stores(c, b):
    bi = wid * rows_per_w + c
    pltpu.async_copy(bufs_s[b].at[pl.ds(0, HALF)],
                     out_hbm.at[bi, pl.ds(0, HALF)], sems_s[b])
    pltpu.async_copy(bufs_s[b].at[pl.ds(HALFP, HALF)],
                     out_hbm.at[bi, pl.ds(HALF, HALF)], sems_s[b])

  def stores_wait(c, b):
    bi = wid * rows_per_w + c
    pltpu.make_async_copy(bufs_s[b].at[pl.ds(0, HALF)],
                          out_hbm.at[bi, pl.ds(0, HALF)], sems_s[b]).wait()
    pltpu.make_async_copy(bufs_s[b].at[pl.ds(HALFP, HALF)],
                          out_hbm.at[bi, pl.ds(HALF, HALF)], sems_s[b]).wait()

  for b in range(NBUF):
    gathers(b, b)

  def loop_body(t, carry):
    for b in range(NBUF):
      c = t * NBUF + b
      gathers_wait(c, b)

      @pl.when(t > 0)
      def _():
        stores_wait(c - NBUF, b)

      @plsc.parallel_loop(0, CHUNK, unroll=4)
      def _(r):
        for cc in range(EMBED_DIM // 16):
          sl = pl.ds(cc * 16, 16)
          bufs_s[b][r, sl] = bufs_g[b][r, sl] * SCALE

      @pl.when(t < nouter - 1)
      def _():
        gathers(c + NBUF, b)

      stores(c, b)
    return carry

  lax.fori_loop(0, nouter, loop_body, 0)

  for b in range(NBUF):
    stores_wait((nouter - 1) * NBUF + b, b)


def _impl(input_ids, table):
  batch, seq_len = input_ids.shape
  vocab, d = table.shape
  rows_per_w = batch // NW

  ids_pad = jnp.pad(
      input_ids.astype(jnp.int32).reshape(batch, 2, HALF),
      ((0, 0), (0, 0), (0, HALFP - HALF)))
  ids3 = ids_pad.reshape(NW, rows_per_w, CHUNK)

  mesh = plsc.VectorSubcoreMesh(core_axis_name="c", subcore_axis_name="s")
  fn = pl.kernel(
      functools.partial(_body, rows_per_w),
      out_type=jax.ShapeDtypeStruct((batch, seq_len, d), jnp.float32),
      mesh=mesh,
      compiler_params=pltpu.CompilerParams(use_tc_tiling_on_sc=False),
      scratch_types=[
          pltpu.VMEM((rows_per_w, CHUNK), jnp.int32),
          [pltpu.VMEM((CHUNK, d), jnp.float32) for _ in range(NBUF)],
          [pltpu.VMEM((CHUNK, d), jnp.float32) for _ in range(NBUF)],
          [pltpu.SemaphoreType.DMA for _ in range(NBUF)],
          [pltpu.SemaphoreType.DMA for _ in range(NBUF)],
      ],
  )
  return fn(ids3, table)


_jitted = None


def kernel(input_ids, table):
  global _jitted
  if _jitted is None:
    fmt = Format(Layout(major_to_minor=(0, 1, 2), tiling=((8,),)),
                 jax.sharding.SingleDeviceSharding(jax.devices()[0]))
    _jitted = jax.jit(_impl, out_shardings=fmt)
  return _jitted(input_ids, table)


# speedup vs baseline: 1.0326x; 1.0326x over previous
import functools

import jax
import jax.numpy as jnp
from jax import lax
from jax.experimental import pallas as pl
from jax.experimental.layout import Format, Layout
from jax.experimental.pallas import tpu as pltpu
from jax.experimental.pallas import tpu_sc as plsc

EMBED_DIM = 64
SCALE = float(EMBED_DIM) ** 0.5

NUM_CORES = 2
NUM_SUBCORES = 16
NW = NUM_CORES * NUM_SUBCORES

HALF = 100
HALFP = 104
CHUNK = 2 * HALFP
NBUFG = 4
NBUFS = 2


def _body(rows_per_w, ids_hbm, table_hbm, out_hbm, idx_v, bufs_g, bufs_s,
          sems_g, sems_s):
  wid = lax.axis_index("s") * NUM_CORES + lax.axis_index("c")
  nchunk = rows_per_w
  nouter = nchunk // NBUFG

  pltpu.sync_copy(ids_hbm.at[wid], idx_v)

  def gathers(c, g):
    pltpu.async_copy(table_hbm.at[idx_v.at[c, pl.ds(0, HALFP)]],
                     bufs_g[g].at[pl.ds(0, HALFP)], sems_g[g])
    pltpu.async_copy(table_hbm.at[idx_v.at[c, pl.ds(HALFP, HALFP)]],
                     bufs_g[g].at[pl.ds(HALFP, HALFP)], sems_g[g])

  def gathers_wait(c, g):
    pltpu.make_async_copy(table_hbm.at[idx_v.at[c, pl.ds(0, HALFP)]],
                          bufs_g[g].at[pl.ds(0, HALFP)], sems_g[g]).wait()
    pltpu.make_async_copy(table_hbm.at[idx_v.at[c, pl.ds(HALFP, HALFP)]],
                          bufs_g[g].at[pl.ds(HALFP, HALFP)], sems_g[g]).wait()

  def stores(c, s):
    bi = wid * rows_per_w + c
    pltpu.async_copy(bufs_s[s].at[pl.ds(0, HALF)],
                     out_hbm.at[bi, pl.ds(0, HALF)], sems_s[s])
    pltpu.async_copy(bufs_s[s].at[pl.ds(HALFP, HALF)],
                     out_hbm.at[bi, pl.ds(HALF, HALF)], sems_s[s])

  def stores_wait(c, s):
    bi = wid * rows_per_w + c
    pltpu.make_async_copy(bufs_s[s].at[pl.ds(0, HALF)],
                          out_hbm.at[bi, pl.ds(0, HALF)], sems_s[s]).wait()
    pltpu.make_async_copy(bufs_s[s].at[pl.ds(HALFP, HALF)],
                          out_hbm.at[bi, pl.ds(HALF, HALF)], sems_s[s]).wait()

  for g in range(NBUFG):
    gathers(g, g)

  def loop_body(t, carry):
    for b in range(NBUFG):
      c = t * NBUFG + b
      s = b % NBUFS
      gathers_wait(c, b)

      if b < NBUFS:
        @pl.when(t > 0)
        def _():
          stores_wait(c - NBUFS, s)
      else:
        stores_wait(c - NBUFS, s)

      @plsc.parallel_loop(0, CHUNK, unroll=4)
      def _(r):
        for cc in range(EMBED_DIM // 16):
          sl = pl.ds(cc * 16, 16)
          bufs_s[s][r, sl] = bufs_g[b][r, sl] * SCALE

      @pl.when(t < nouter - 1)
      def _():
        gathers(c + NBUFG, b)

      stores(c, s)
    return carry

  lax.fori_loop(0, nouter, loop_body, 0)

  for s in range(NBUFS):
    stores_wait((nouter - 1) * NBUFG + NBUFG - NBUFS + s, s)


def _impl(input_ids, table):
  batch, seq_len = input_ids.shape
  vocab, d = table.shape
  rows_per_w = batch // NW

  table2 = jnp.pad(table, ((0, 0), (0, 128 - d))).reshape(2 * vocab, d)

  ids_pad = jnp.pad(
      (input_ids.astype(jnp.int32) << 1).reshape(batch, 2, HALF),
      ((0, 0), (0, 0), (0, HALFP - HALF)))
  ids3 = ids_pad.reshape(NW, rows_per_w, CHUNK)

  mesh = plsc.VectorSubcoreMesh(core_axis_name="c", subcore_axis_name="s")
  fn = pl.kernel(
      functools.partial(_body, rows_per_w),
      out_type=jax.ShapeDtypeStruct((batch, seq_len, d), jnp.float32),
      mesh=mesh,
      compiler_params=pltpu.CompilerParams(use_tc_tiling_on_sc=False),
      scratch_types=[
          pltpu.VMEM((rows_per_w, CHUNK), jnp.int32),
          [pltpu.VMEM((CHUNK, d), jnp.float32) for _ in range(NBUFG)],
          [pltpu.VMEM((CHUNK, d), jnp.float32) for _ in range(NBUFS)],
          [pltpu.SemaphoreType.DMA for _ in range(NBUFG)],
          [pltpu.SemaphoreType.DMA for _ in range(NBUFS)],
      ],
  )
  out = fn(ids3, table2)
  from jax.experimental.layout import with_layout_constraint
  return with_layout_constraint(
      out, Layout(major_to_minor=(0, 1, 2), tiling=((8,),)))


_jitted = None


def kernel(input_ids, table):
  global _jitted
  if _jitted is None:
    fmt = Format(Layout(major_to_minor=(0, 1, 2), tiling=((8,),)),
                 jax.sharding.SingleDeviceSharding(jax.devices()[0]))
    _jitted = jax.jit(_impl, out_shardings=fmt)
  return _jitted(input_ids, table)


# speedup vs baseline: 1.6749x; 1.6220x over previous
import functools

import jax
import jax.numpy as jnp
from jax import lax
from jax.experimental import pallas as pl
from jax.experimental.pallas import tpu as pltpu
from jax.experimental.pallas import tpu_sc as plsc

EMBED_DIM = 64
SCALE = float(EMBED_DIM) ** 0.5

NUM_CORES = 1
NUM_SUBCORES = 16
NW = NUM_CORES * NUM_SUBCORES

CHUNK = 128
NBUF = 4


def _body(nchunk, rows_per_w, ids_hbm, table_hbm, out_hbm,
          idx_v, bufs_g, bufs_s, sems_g, sems_s):
  wid = lax.axis_index("s") * NUM_CORES + lax.axis_index("c")
  base = wid * rows_per_w
  nouter = nchunk // NBUF

  pltpu.sync_copy(ids_hbm.at[wid], idx_v)

  def gather(c, b):
    return pltpu.async_copy(table_hbm.at[idx_v.at[c]], bufs_g[b], sems_g[b])

  def gather_wait(c, b):
    pltpu.make_async_copy(table_hbm.at[idx_v.at[c]], bufs_g[b],
                          sems_g[b]).wait()

  def store(c, b):
    return pltpu.async_copy(
        bufs_s[b], out_hbm.at[pl.ds(base + c * CHUNK, CHUNK)], sems_s[b])

  def store_wait(c, b):
    pltpu.make_async_copy(
        bufs_s[b], out_hbm.at[pl.ds(base + c * CHUNK, CHUNK)],
        sems_s[b]).wait()

  for b in range(NBUF):
    gather(b, b)

  def loop_body(t, carry):
    for b in range(NBUF):
      c = t * NBUF + b
      gather_wait(c, b)

      @pl.when(t > 0)
      def _():
        store_wait(c - NBUF, b)

      @plsc.parallel_loop(0, CHUNK, unroll=4)
      def _(r):
        for cc in range(EMBED_DIM // 16):
          sl = pl.ds(cc * 16, 16)
          bufs_s[b][r, sl] = bufs_g[b][r, sl] * SCALE

      store(c, b)

      @pl.when(t < nouter - 1)
      def _():
        gather(c + NBUF, b)
    return carry

  lax.fori_loop(0, nouter, loop_body, 0)

  for b in range(NBUF):
    store_wait((nouter - 1) * NBUF + b, b)


@jax.jit
def kernel(input_ids, table):
  batch, seq_len = input_ids.shape
  vocab, d = table.shape
  n = batch * seq_len
  rows_per_w = n // NW
  nchunk = rows_per_w // CHUNK

  ids3 = input_ids.reshape(NW, nchunk, CHUNK).astype(jnp.int32)

  mesh = plsc.VectorSubcoreMesh(core_axis_name="c", subcore_axis_name="s",
                                num_cores=NUM_CORES)
  fn = pl.kernel(
      functools.partial(_body, nchunk, rows_per_w),
      out_type=jax.ShapeDtypeStruct((n, d), jnp.float32),
      mesh=mesh,
      compiler_params=pltpu.CompilerParams(use_tc_tiling_on_sc=False),
      scratch_types=[
          pltpu.VMEM((nchunk, CHUNK), jnp.int32),
          [pltpu.VMEM((CHUNK, d), jnp.float32) for _ in range(NBUF)],
          [pltpu.VMEM((CHUNK, d), jnp.float32) for _ in range(NBUF)],
          [pltpu.SemaphoreType.DMA for _ in range(NBUF)],
          [pltpu.SemaphoreType.DMA for _ in range(NBUF)],
      ],
  )
  out = fn(ids3, table)
  return out.reshape(batch, seq_len, d)


# speedup vs baseline: 1.7192x; 1.0264x over previous
import functools

import jax
import jax.numpy as jnp
from jax import lax
from jax.experimental import pallas as pl
from jax.experimental.pallas import tpu as pltpu
from jax.experimental.pallas import tpu_sc as plsc

EMBED_DIM = 64
SCALE = float(EMBED_DIM) ** 0.5

NUM_CORES = 2
NUM_SUBCORES = 16
NW = NUM_CORES * NUM_SUBCORES

CHUNK = 128
NBUF = 4


def _body(nchunk, rows_per_w, ids_hbm, table_hbm, out_hbm,
          idx_v, bufs_g, bufs_s, sems_g, sems_s):
  wid = lax.axis_index("s") * NUM_CORES + lax.axis_index("c")
  base = wid * rows_per_w
  nouter = nchunk // NBUF

  pltpu.sync_copy(ids_hbm.at[wid], idx_v)

  def gather(c, b):
    return pltpu.async_copy(table_hbm.at[idx_v.at[c]], bufs_g[b], sems_g[b])

  def gather_wait(c, b):
    pltpu.make_async_copy(table_hbm.at[idx_v.at[c]], bufs_g[b],
                          sems_g[b]).wait()

  def store(c, b):
    return pltpu.async_copy(
        bufs_s[b], out_hbm.at[pl.ds(base + c * CHUNK, CHUNK)], sems_s[b])

  def store_wait(c, b):
    pltpu.make_async_copy(
        bufs_s[b], out_hbm.at[pl.ds(base + c * CHUNK, CHUNK)],
        sems_s[b]).wait()

  for b in range(NBUF):
    gather(b, b)

  def loop_body(t, carry):
    for b in range(NBUF):
      c = t * NBUF + b
      gather_wait(c, b)

      @pl.when(t > 0)
      def _():
        store_wait(c - NBUF, b)

      @plsc.parallel_loop(0, CHUNK, unroll=4)
      def _(r):
        for cc in range(EMBED_DIM // 16):
          sl = pl.ds(cc * 16, 16)
          bufs_s[b][r, sl] = bufs_g[b][r, sl] * SCALE

      store(c, b)

      @pl.when(t < nouter - 1)
      def _():
        gather(c + NBUF, b)
    return carry

  lax.fori_loop(0, nouter, loop_body, 0)

  for b in range(NBUF):
    store_wait((nouter - 1) * NBUF + b, b)


@jax.jit
def kernel(input_ids, table):
  batch, seq_len = input_ids.shape
  vocab, d = table.shape
  n = batch * seq_len
  rows_per_w = n // NW
  nchunk = rows_per_w // CHUNK

  ids3 = input_ids.reshape(NW, nchunk, CHUNK).astype(jnp.int32)

  mesh = plsc.VectorSubcoreMesh(core_axis_name="c", subcore_axis_name="s",
                                num_cores=NUM_CORES)
  fn = pl.kernel(
      functools.partial(_body, nchunk, rows_per_w),
      out_type=jax.ShapeDtypeStruct((n, d), jnp.float32),
      mesh=mesh,
      compiler_params=pltpu.CompilerParams(use_tc_tiling_on_sc=False),
      scratch_types=[
          pltpu.VMEM((nchunk, CHUNK), jnp.int32),
          [pltpu.VMEM((CHUNK, d), jnp.float32) for _ in range(NBUF)],
          [pltpu.VMEM((CHUNK, d), jnp.float32) for _ in range(NBUF)],
          [pltpu.SemaphoreType.DMA for _ in range(NBUF)],
          [pltpu.SemaphoreType.DMA for _ in range(NBUF)],
      ],
  )
  out = fn(ids3, table)
  return out.reshape(batch, seq_len, d)


# speedup vs baseline: 1.7807x; 1.0358x over previous
import functools

import jax
import jax.numpy as jnp
from jax import lax
from jax.experimental import pallas as pl
from jax.experimental.pallas import tpu as pltpu
from jax.experimental.pallas import tpu_sc as plsc

EMBED_DIM = 64
SCALE = float(EMBED_DIM) ** 0.5

NUM_CORES = 2
NUM_SUBCORES = 16
NW = NUM_CORES * NUM_SUBCORES

CHUNK = 128
NBUFG = 3
NBUFS = 2


def _body(seq_len, ids_hbm, table_hbm, out_hbm, idx_v, bufs_g, bufs_s,
          sems_g, sems_s):
  wid = lax.axis_index("s") * NUM_CORES + lax.axis_index("c")
  b0 = wid * CHUNK

  pltpu.sync_copy(ids_hbm.at[wid], idx_v)

  def gather(c, g):
    return pltpu.async_copy(table_hbm.at[idx_v.at[c]], bufs_g[g], sems_g[g])

  def gather_wait(c, g):
    pltpu.make_async_copy(table_hbm.at[idx_v.at[c]], bufs_g[g],
                          sems_g[g]).wait()

  def store(c, s):
    return pltpu.async_copy(bufs_s[s], out_hbm.at[c, :, pl.ds(b0, CHUNK)],
                            sems_s[s])

  def store_wait(c, s):
    pltpu.make_async_copy(bufs_s[s], out_hbm.at[c, :, pl.ds(b0, CHUNK)],
                          sems_s[s]).wait()

  for g in range(NBUFG):
    gather(g, g)

  lanes = lax.iota(jnp.int32, 16)

  def chunk_body(c, g, s, first, last):
    gather_wait(c, g)

    if first:
      @pl.when(c >= NBUFS)
      def _():
        store_wait(c - NBUFS, s)
    else:
      store_wait(c - NBUFS, s)

    @plsc.parallel_loop(0, EMBED_DIM, unroll=2)
    def _(d):
      dvec = jnp.full((16,), 0, jnp.int32) + d
      for t0 in range(CHUNK // 16):
        vals = plsc.load_gather(bufs_g[g], [lanes + (16 * t0), dvec])
        bufs_s[s][d, pl.ds(16 * t0, 16)] = vals * SCALE

    if not last:
      gather(c + NBUFG, g)
    store(c, s)

  def loop_body(t, carry):
    for b in range(NBUFG * NBUFS):
      c = t * (NBUFG * NBUFS) + b
      chunk_body(c, b % NBUFG, b % NBUFS, first=(b < NBUFS), last=False)
    return carry

  nmain = (seq_len - NBUFG) // (NBUFG * NBUFS)
  lax.fori_loop(0, nmain, loop_body, 0)
  for b in range(seq_len - nmain * NBUFG * NBUFS):
    c = nmain * NBUFG * NBUFS + b
    chunk_body(c, c % NBUFG, c % NBUFS, first=False,
               last=(c + NBUFG >= seq_len))

  for k in range(NBUFS):
    c = seq_len - NBUFS + k
    store_wait(c, c % NBUFS)


@jax.jit
def kernel(input_ids, table):
  batch, seq_len = input_ids.shape
  vocab, d = table.shape

  table2 = jnp.pad(table, ((0, 0), (0, 128 - d)))
  ids3 = (input_ids.astype(jnp.int32).T
          .reshape(seq_len, NW, CHUNK).transpose(1, 0, 2))

  mesh = plsc.VectorSubcoreMesh(core_axis_name="c", subcore_axis_name="s")
  fn = pl.kernel(
      functools.partial(_body, seq_len),
      out_type=jax.ShapeDtypeStruct((seq_len, d, batch), jnp.float32),
      mesh=mesh,
      compiler_params=pltpu.CompilerParams(use_tc_tiling_on_sc=True,
                                           needs_layout_passes=False),
      scratch_types=[
          pltpu.VMEM((seq_len, CHUNK), jnp.int32),
          [pltpu.VMEM((CHUNK, 128), jnp.float32) for _ in range(NBUFG)],
          [pltpu.VMEM((d, CHUNK), jnp.float32) for _ in range(NBUFS)],
          [pltpu.SemaphoreType.DMA for _ in range(NBUFG)],
          [pltpu.SemaphoreType.DMA for _ in range(NBUFS)],
      ],
  )
  out = fn(ids3, table2)
  return jnp.transpose(out, (2, 0, 1))


# speedup vs baseline: 1.7816x; 1.0005x over previous
import functools

import jax
import jax.numpy as jnp
from jax import lax
from jax.experimental import pallas as pl
from jax.experimental.pallas import tpu as pltpu
from jax.experimental.pallas import tpu_sc as plsc

EMBED_DIM = 64
SCALE = float(EMBED_DIM) ** 0.5

NUM_CORES = 2
NUM_SUBCORES = 16
NW = NUM_CORES * NUM_SUBCORES

CHUNK = 128
NBUFG = 3
NBUFS = 2


def _body(seq_len, ids_hbm, table_hbm, out_hbm, idx_v, bufs_g, bufs_s,
          sems_g, sems_s):
  wid = lax.axis_index("s") * NUM_CORES + lax.axis_index("c")
  b0 = wid * CHUNK

  pltpu.sync_copy(ids_hbm.at[wid], idx_v)

  def gather(c, g):
    return pltpu.async_copy(table_hbm.at[idx_v.at[c]], bufs_g[g], sems_g[g])

  def gather_wait(c, g):
    pltpu.make_async_copy(table_hbm.at[idx_v.at[c]], bufs_g[g],
                          sems_g[g]).wait()

  def store(c, s):
    return pltpu.async_copy(bufs_s[s], out_hbm.at[c, :, pl.ds(b0, CHUNK)],
                            sems_s[s])

  def store_wait(c, s):
    pltpu.make_async_copy(bufs_s[s], out_hbm.at[c, :, pl.ds(b0, CHUNK)],
                          sems_s[s]).wait()

  for g in range(NBUFG):
    gather(g, g)

  lanes = lax.iota(jnp.int32, 16)
  zeros16 = lanes * 0
  tvecs = [lanes + (16 * t0) for t0 in range(CHUNK // 16)]

  def chunk_body(c, g, s, first, last):
    gather_wait(c, g)

    if first:
      @pl.when(c >= NBUFS)
      def _():
        store_wait(c - NBUFS, s)
    else:
      store_wait(c - NBUFS, s)

    @plsc.parallel_loop(0, EMBED_DIM, unroll=4)
    def _(d):
      dvec = zeros16 + d
      for t0 in range(CHUNK // 16):
        vals = plsc.load_gather(bufs_g[g], [tvecs[t0], dvec])
        bufs_s[s][d, pl.ds(16 * t0, 16)] = vals * SCALE

    if not last:
      gather(c + NBUFG, g)
    store(c, s)

  def loop_body(t, carry):
    for b in range(NBUFG * NBUFS):
      c = t * (NBUFG * NBUFS) + b
      chunk_body(c, b % NBUFG, b % NBUFS, first=(b < NBUFS), last=False)
    return carry

  nmain = (seq_len - NBUFG) // (NBUFG * NBUFS)
  lax.fori_loop(0, nmain, loop_body, 0)
  for b in range(seq_len - nmain * NBUFG * NBUFS):
    c = nmain * NBUFG * NBUFS + b
    chunk_body(c, c % NBUFG, c % NBUFS, first=False,
               last=(c + NBUFG >= seq_len))

  for k in range(NBUFS):
    c = seq_len - NBUFS + k
    store_wait(c, c % NBUFS)


@jax.jit
def kernel(input_ids, table):
  batch, seq_len = input_ids.shape
  vocab, d = table.shape

  table2 = jnp.pad(table, ((0, 0), (0, 128 - d)))
  ids3 = (input_ids.astype(jnp.int32).T
          .reshape(seq_len, NW, CHUNK).transpose(1, 0, 2))

  mesh = plsc.VectorSubcoreMesh(core_axis_name="c", subcore_axis_name="s")
  fn = pl.kernel(
      functools.partial(_body, seq_len),
      out_type=jax.ShapeDtypeStruct((seq_len, d, batch), jnp.float32),
      mesh=mesh,
      compiler_params=pltpu.CompilerParams(use_tc_tiling_on_sc=True,
                                           needs_layout_passes=False),
      scratch_types=[
          pltpu.VMEM((seq_len, CHUNK), jnp.int32),
          [pltpu.VMEM((CHUNK, 128), jnp.float32) for _ in range(NBUFG)],
          [pltpu.VMEM((d, CHUNK), jnp.float32) for _ in range(NBUFS)],
          [pltpu.SemaphoreType.DMA for _ in range(NBUFG)],
          [pltpu.SemaphoreType.DMA for _ in range(NBUFS)],
      ],
  )
  out = fn(ids3, table2)
  return jnp.transpose(out, (2, 0, 1))
